# baseline (speedup 1.0000x reference)
import jax
import jax.numpy as jnp
from jax import lax
from jax.experimental import pallas as pl
from jax.experimental.pallas import tpu as pltpu
from jax.experimental.pallas import tpu_sc as plsc

V = 1024
N = 32768
D = 256

BN = 4096
NT = 16384
NSC = N - NT

NC = 2
NS = 16
DC = D // NC
BLK = 128
ROWS_PER_TILE = NSC // NS
NBLK = ROWS_PER_TILE // BLK
NBUF = min(4, NBLK)
IDX_ROWS = N // BLK


def _sc_body(h_hbm, idx_hbm, z_hbm, out_hbm, *refs):
    bufs = list(refs[0:NBUF])
    idx2, acc = refs[NBUF], refs[NBUF + 1]
    gsem = list(refs[NBUF + 2:NBUF + 2 + NBUF])
    ssem = list(refs[NBUF + 2 + NBUF:NBUF + 2 + 2 * NBUF])

    c = lax.axis_index("c")
    s = lax.axis_index("s")
    row0 = NT + s * ROWS_PER_TILE
    col0 = c * DC
    rpt = V // NS

    pltpu.sync_copy(z_hbm, acc.at[pl.ds(s * rpt, rpt)])
    pltpu.sync_copy(
        idx_hbm.at[pl.ds(NT // BLK + s * NBLK, NBLK)], idx2
    )

    def gather(b):
        return pltpu.async_copy(
            h_hbm.at[pl.ds(row0 + b * BLK, BLK), pl.ds(col0, DC)],
            bufs[b % NBUF],
            gsem[b % NBUF],
        )

    gath = [None] * NBLK
    scat = [None] * NBLK
    for b in range(min(2, NBLK)):
        gath[b] = gather(b)

    plsc.subcore_barrier()

    for b in range(NBLK):
        nb = b + 2
        if nb < NBLK:
            if nb - NBUF >= 0:
                scat[nb - NBUF].wait()
            gath[nb] = gather(nb)
        gath[b].wait()
        scat[b] = pltpu.async_copy(
            bufs[b % NBUF], acc.at[idx2.at[b]], ssem[b % NBUF], add=True
        )
    for b in range(max(0, NBLK - NBUF), NBLK):
        scat[b].wait()

    plsc.subcore_barrier()

    pltpu.sync_copy(
        acc.at[pl.ds(s * rpt, rpt)],
        out_hbm.at[pl.ds(s * rpt, rpt), pl.ds(col0, DC)],
    )


def _sc_aggr(H, idx2d, zeros):
    mesh = plsc.VectorSubcoreMesh(core_axis_name="c", subcore_axis_name="s")
    f = pl.kernel(
        _sc_body,
        out_type=jax.ShapeDtypeStruct((V, D), jnp.float32),
        mesh=mesh,
        scratch_types=(
            [pltpu.VMEM((BLK, DC), jnp.float32) for _ in range(NBUF)]
            + [
                pltpu.VMEM((NBLK, BLK), jnp.int32),
                pltpu.VMEM_SHARED((V, DC), jnp.float32),
            ]
            + [pltpu.SemaphoreType.DMA for _ in range(2 * NBUF)]
        ),
    )
    return f(H, idx2d, zeros)


def _mm_body(idx_ref, h_ref, o_ref):
    i = pl.program_id(0)

    @pl.when(i == 0)
    def _():
        o_ref[...] = jnp.zeros_like(o_ref)

    idx = idx_ref[0, 0, :].astype(jnp.int16)
    iota = lax.broadcasted_iota(jnp.int16, (V, BN), 0)
    mask = jnp.where(
        iota == idx[None, :], jnp.bfloat16(1.0), jnp.bfloat16(0.0)
    )
    o_ref[...] += jnp.dot(
        mask, h_ref[...].astype(jnp.bfloat16), preferred_element_type=jnp.float32
    )


def _tc_matmul(H, idx3):
    return pl.pallas_call(
        _mm_body,
        grid=(NT // BN,),
        in_specs=[
            pl.BlockSpec((1, 1, BN), lambda i: (i, 0, 0)),
            pl.BlockSpec((BN, D), lambda i: (i, 0)),
        ],
        out_specs=pl.BlockSpec((V, D), lambda i: (0, 0)),
        out_shape=jax.ShapeDtypeStruct((V, D), jnp.float32),
        compiler_params=pltpu.CompilerParams(
            dimension_semantics=("arbitrary",),
        ),
    )(idx3, H)


def _add_body(a_ref, b_ref, o_ref):
    o_ref[...] = a_ref[...] + b_ref[...]


def _add(a, b):
    bv = V // 8
    return pl.pallas_call(
        _add_body,
        grid=(8,),
        in_specs=[
            pl.BlockSpec((bv, D), lambda i: (i, 0)),
            pl.BlockSpec((bv, D), lambda i: (i, 0)),
        ],
        out_specs=pl.BlockSpec((bv, D), lambda i: (i, 0)),
        out_shape=jax.ShapeDtypeStruct((V, D), jnp.float32),
        compiler_params=pltpu.CompilerParams(
            dimension_semantics=("arbitrary",),
        ),
    )(a, b)


@jax.jit
def kernel(H, X_node):
    idx2d = X_node.reshape(IDX_ROWS, BLK)
    idx3 = X_node.reshape(N // BN, 1, BN)
    zeros = jnp.zeros((V // NS, DC), jnp.float32)
    sc_part = _sc_aggr(H, idx2d, zeros)
    tc_part = _tc_matmul(H, idx3)
    return _add(tc_part, sc_part)


# speedup vs baseline: 1.0413x; 1.0413x over previous
import jax
import jax.numpy as jnp
from jax import lax
from jax.experimental import pallas as pl
from jax.experimental.pallas import tpu as pltpu
from jax.experimental.pallas import tpu_sc as plsc

V = 1024
N = 32768
D = 256

BN = 4096
NT = 24576
NSC = N - NT

NC = 2
NS = 16
DC = D // NC
BLK = 128
ROWS_PER_TILE = NSC // NS
NBLK = ROWS_PER_TILE // BLK
NBUF = min(4, NBLK)
IDX_ROWS = N // BLK


def _sc_body(h_hbm, idx_hbm, z_hbm, out_hbm, *refs):
    bufs = list(refs[0:NBUF])
    idx2, acc = refs[NBUF], refs[NBUF + 1]
    gsem = list(refs[NBUF + 2:NBUF + 2 + NBUF])
    ssem = list(refs[NBUF + 2 + NBUF:NBUF + 2 + 2 * NBUF])

    c = lax.axis_index("c")
    s = lax.axis_index("s")
    row0 = NT + s * ROWS_PER_TILE
    col0 = c * DC
    rpt = V // NS

    pltpu.sync_copy(z_hbm, acc.at[pl.ds(s * rpt, rpt)])
    pltpu.sync_copy(
        idx_hbm.at[pl.ds(NT // BLK + s * NBLK, NBLK)], idx2
    )

    def gather(b):
        return pltpu.async_copy(
            h_hbm.at[pl.ds(row0 + b * BLK, BLK), pl.ds(col0, DC)],
            bufs[b % NBUF],
            gsem[b % NBUF],
        )

    gath = [None] * NBLK
    scat = [None] * NBLK
    for b in range(min(2, NBLK)):
        gath[b] = gather(b)

    plsc.subcore_barrier()

    for b in range(NBLK):
        nb = b + 2
        if nb < NBLK:
            if nb - NBUF >= 0:
                scat[nb - NBUF].wait()
            gath[nb] = gather(nb)
        gath[b].wait()
        scat[b] = pltpu.async_copy(
            bufs[b % NBUF], acc.at[idx2.at[b]], ssem[b % NBUF], add=True
        )
    for b in range(max(0, NBLK - NBUF), NBLK):
        scat[b].wait()

    plsc.subcore_barrier()

    pltpu.sync_copy(
        acc.at[pl.ds(s * rpt, rpt)],
        out_hbm.at[pl.ds(s * rpt, rpt), pl.ds(col0, DC)],
    )


def _sc_aggr(H, idx2d, zeros):
    mesh = plsc.VectorSubcoreMesh(core_axis_name="c", subcore_axis_name="s")
    f = pl.kernel(
        _sc_body,
        out_type=jax.ShapeDtypeStruct((V, D), jnp.float32),
        mesh=mesh,
        scratch_types=(
            [pltpu.VMEM((BLK, DC), jnp.float32) for _ in range(NBUF)]
            + [
                pltpu.VMEM((NBLK, BLK), jnp.int32),
                pltpu.VMEM_SHARED((V, DC), jnp.float32),
            ]
            + [pltpu.SemaphoreType.DMA for _ in range(2 * NBUF)]
        ),
    )
    return f(H, idx2d, zeros)


def _mm_body(idx_ref, h_ref, o_ref):
    i = pl.program_id(0)

    @pl.when(i == 0)
    def _():
        o_ref[...] = jnp.zeros_like(o_ref)

    idx = idx_ref[0, 0, :].astype(jnp.int16)
    iota = lax.broadcasted_iota(jnp.int16, (V, BN), 0)
    mask = jnp.where(
        iota == idx[None, :], jnp.bfloat16(1.0), jnp.bfloat16(0.0)
    )
    o_ref[...] += jnp.dot(
        mask, h_ref[...].astype(jnp.bfloat16), preferred_element_type=jnp.float32
    )


def _tc_matmul(H, idx3):
    return pl.pallas_call(
        _mm_body,
        grid=(NT // BN,),
        in_specs=[
            pl.BlockSpec((1, 1, BN), lambda i: (i, 0, 0)),
            pl.BlockSpec((BN, D), lambda i: (i, 0)),
        ],
        out_specs=pl.BlockSpec((V, D), lambda i: (0, 0)),
        out_shape=jax.ShapeDtypeStruct((V, D), jnp.float32),
        compiler_params=pltpu.CompilerParams(
            dimension_semantics=("arbitrary",),
        ),
    )(idx3, H)


def _add_body(a_ref, b_ref, o_ref):
    o_ref[...] = a_ref[...] + b_ref[...]


def _add(a, b):
    bv = V // 8
    return pl.pallas_call(
        _add_body,
        grid=(8,),
        in_specs=[
            pl.BlockSpec((bv, D), lambda i: (i, 0)),
            pl.BlockSpec((bv, D), lambda i: (i, 0)),
        ],
        out_specs=pl.BlockSpec((bv, D), lambda i: (i, 0)),
        out_shape=jax.ShapeDtypeStruct((V, D), jnp.float32),
        compiler_params=pltpu.CompilerParams(
            dimension_semantics=("arbitrary",),
        ),
    )(a, b)


@jax.jit
def kernel(H, X_node):
    idx2d = X_node.reshape(IDX_ROWS, BLK)
    idx3 = X_node.reshape(N // BN, 1, BN)
    zeros = jnp.zeros((V // NS, DC), jnp.float32)
    sc_part = _sc_aggr(H, idx2d, zeros)
    tc_part = _tc_matmul(H, idx3)
    return _add(tc_part, sc_part)


# speedup vs baseline: 1.1231x; 1.0785x over previous
import jax
import jax.numpy as jnp
from jax import lax
from jax.experimental import pallas as pl
from jax.experimental.pallas import tpu as pltpu
from jax.experimental.pallas import tpu_sc as plsc

V = 1024
N = 32768
D = 256

BN = 4096
NT = 24576
NSC = N - NT

NC = 2
NS = 16
DC = D // NC
BLK = 128
ROWS_PER_TILE = NSC // NS
NBLK = ROWS_PER_TILE // BLK
NBUF = min(4, NBLK)
IDX_ROWS = N // BLK


def _sc_body(h_hbm, idx_hbm, z_hbm, out_hbm, *refs):
    bufs = list(refs[0:NBUF])
    idx2, acc = refs[NBUF], refs[NBUF + 1]
    gsem = list(refs[NBUF + 2:NBUF + 2 + NBUF])
    ssem = list(refs[NBUF + 2 + NBUF:NBUF + 2 + 2 * NBUF])

    c = lax.axis_index("c")
    s = lax.axis_index("s")
    row0 = NT + s * ROWS_PER_TILE
    col0 = c * DC
    rpt = V // NS

    pltpu.sync_copy(z_hbm, acc.at[pl.ds(s * rpt, rpt)])
    pltpu.sync_copy(
        idx_hbm.at[pl.ds(NT // BLK + s * NBLK, NBLK)], idx2
    )

    def gather(b):
        return pltpu.async_copy(
            h_hbm.at[pl.ds(row0 + b * BLK, BLK), pl.ds(col0, DC)],
            bufs[b % NBUF],
            gsem[b % NBUF],
        )

    gath = [None] * NBLK
    scat = [None] * NBLK
    for b in range(min(2, NBLK)):
        gath[b] = gather(b)

    plsc.subcore_barrier()

    for b in range(NBLK):
        nb = b + 2
        if nb < NBLK:
            if nb - NBUF >= 0:
                scat[nb - NBUF].wait()
            gath[nb] = gather(nb)
        gath[b].wait()
        scat[b] = pltpu.async_copy(
            bufs[b % NBUF], acc.at[idx2.at[b]], ssem[b % NBUF], add=True
        )
    for b in range(max(0, NBLK - NBUF), NBLK):
        scat[b].wait()

    plsc.subcore_barrier()

    pltpu.sync_copy(
        acc.at[pl.ds(s * rpt, rpt)],
        out_hbm.at[pl.ds(s * rpt, rpt), pl.ds(col0, DC)],
    )


def _sc_aggr(H, idx2d, zeros):
    mesh = plsc.VectorSubcoreMesh(core_axis_name="c", subcore_axis_name="s")
    f = pl.kernel(
        _sc_body,
        out_type=jax.ShapeDtypeStruct((V, D), jnp.float32),
        mesh=mesh,
        scratch_types=(
            [pltpu.VMEM((BLK, DC), jnp.float32) for _ in range(NBUF)]
            + [
                pltpu.VMEM((NBLK, BLK), jnp.int32),
                pltpu.VMEM_SHARED((V, DC), jnp.float32),
            ]
            + [pltpu.SemaphoreType.DMA for _ in range(2 * NBUF)]
        ),
    )
    return f(H, idx2d, zeros)


def _mm_body(idx_ref, h_ref, o_ref):
    i = pl.program_id(0)

    @pl.when(i == 0)
    def _():
        o_ref[...] = jnp.zeros_like(o_ref)

    idx = idx_ref[0, 0, :].astype(jnp.int16)
    iota = lax.broadcasted_iota(jnp.int16, (V, BN), 0)
    mask = jnp.where(
        iota == idx[None, :], jnp.bfloat16(1.0), jnp.bfloat16(0.0)
    )
    o_ref[...] += jnp.dot(
        mask, h_ref[...].astype(jnp.bfloat16), preferred_element_type=jnp.float32
    )


def _tc_matmul(H, idx3):
    return pl.pallas_call(
        _mm_body,
        grid=(NT // BN,),
        in_specs=[
            pl.BlockSpec((1, 1, BN), lambda i: (i, 0, 0)),
            pl.BlockSpec((BN, D), lambda i: (i, 0)),
        ],
        out_specs=pl.BlockSpec((V, D), lambda i: (0, 0)),
        out_shape=jax.ShapeDtypeStruct((V, D), jnp.float32),
        compiler_params=pltpu.CompilerParams(
            dimension_semantics=("arbitrary",),
        ),
    )(idx3, H)


def _add_body(a_ref, b_ref, o_ref):
    o_ref[...] = a_ref[...] + b_ref[...]


def _add(a, b):
    return pl.pallas_call(
        _add_body,
        out_shape=jax.ShapeDtypeStruct((V, D), jnp.float32),
    )(a, b)


@jax.jit
def kernel(H, X_node):
    idx2d = X_node.reshape(IDX_ROWS, BLK)
    idx3 = X_node.reshape(N // BN, 1, BN)
    zeros = jnp.zeros((V // NS, DC), jnp.float32)
    sc_part = _sc_aggr(H, idx2d, zeros)
    tc_part = _tc_matmul(H, idx3)
    return _add(tc_part, sc_part)


# speedup vs baseline: 1.1488x; 1.0229x over previous
import jax
import jax.numpy as jnp
from jax import lax
from jax.experimental import pallas as pl
from jax.experimental.pallas import tpu as pltpu
from jax.experimental.pallas import tpu_sc as plsc

V = 1024
N = 32768
D = 256

BN = 4096
NT = 20480
NSC = N - NT

NC = 2
NS = 16
DC = D // NC
BLK = 128
ROWS_PER_TILE = NSC // NS
NBLK = ROWS_PER_TILE // BLK
NBUF = min(4, NBLK)
IDX_ROWS = N // BLK


def _sc_body(h_hbm, idx_hbm, z_hbm, out_hbm, *refs):
    bufs = list(refs[0:NBUF])
    idx2, acc = refs[NBUF], refs[NBUF + 1]
    gsem = list(refs[NBUF + 2:NBUF + 2 + NBUF])
    ssem = list(refs[NBUF + 2 + NBUF:NBUF + 2 + 2 * NBUF])

    c = lax.axis_index("c")
    s = lax.axis_index("s")
    row0 = NT + s * ROWS_PER_TILE
    col0 = c * DC
    rpt = V // NS

    pltpu.sync_copy(z_hbm, acc.at[pl.ds(s * rpt, rpt)])
    start = NT // BLK + s * NBLK
    win = (start // 8) * 8
    delta = start - win
    pltpu.sync_copy(idx_hbm.at[pl.ds(win, 16)], idx2)

    def gather(b):
        return pltpu.async_copy(
            h_hbm.at[pl.ds(row0 + b * BLK, BLK), pl.ds(col0, DC)],
            bufs[b % NBUF],
            gsem[b % NBUF],
        )

    gath = [None] * NBLK
    scat = [None] * NBLK
    for b in range(min(2, NBLK)):
        gath[b] = gather(b)

    plsc.subcore_barrier()

    for b in range(NBLK):
        nb = b + 2
        if nb < NBLK:
            if nb - NBUF >= 0:
                scat[nb - NBUF].wait()
            gath[nb] = gather(nb)
        gath[b].wait()
        scat[b] = pltpu.async_copy(
            bufs[b % NBUF], acc.at[idx2.at[delta + b]], ssem[b % NBUF], add=True
        )
    for b in range(max(0, NBLK - NBUF), NBLK):
        scat[b].wait()

    plsc.subcore_barrier()

    pltpu.sync_copy(
        acc.at[pl.ds(s * rpt, rpt)],
        out_hbm.at[pl.ds(s * rpt, rpt), pl.ds(col0, DC)],
    )


def _sc_aggr(H, idx2d, zeros):
    mesh = plsc.VectorSubcoreMesh(core_axis_name="c", subcore_axis_name="s")
    f = pl.kernel(
        _sc_body,
        out_type=jax.ShapeDtypeStruct((V, D), jnp.float32),
        mesh=mesh,
        scratch_types=(
            [pltpu.VMEM((BLK, DC), jnp.float32) for _ in range(NBUF)]
            + [
                pltpu.VMEM((16, BLK), jnp.int32),
                pltpu.VMEM_SHARED((V, DC), jnp.float32),
            ]
            + [pltpu.SemaphoreType.DMA for _ in range(2 * NBUF)]
        ),
    )
    return f(H, idx2d, zeros)


def _mm_body(idx_ref, h_ref, o_ref):
    i = pl.program_id(0)

    @pl.when(i == 0)
    def _():
        o_ref[...] = jnp.zeros_like(o_ref)

    idx = idx_ref[0, 0, :].astype(jnp.int16)
    iota = lax.broadcasted_iota(jnp.int16, (V, BN), 0)
    mask = jnp.where(
        iota == idx[None, :], jnp.bfloat16(1.0), jnp.bfloat16(0.0)
    )
    o_ref[...] += jnp.dot(
        mask, h_ref[...].astype(jnp.bfloat16), preferred_element_type=jnp.float32
    )


def _tc_matmul(H, idx3):
    return pl.pallas_call(
        _mm_body,
        grid=(NT // BN,),
        in_specs=[
            pl.BlockSpec((1, 1, BN), lambda i: (i, 0, 0)),
            pl.BlockSpec((BN, D), lambda i: (i, 0)),
        ],
        out_specs=pl.BlockSpec((V, D), lambda i: (0, 0)),
        out_shape=jax.ShapeDtypeStruct((V, D), jnp.float32),
        compiler_params=pltpu.CompilerParams(
            dimension_semantics=("arbitrary",),
        ),
    )(idx3, H)


def _add_body(a_ref, b_ref, o_ref):
    o_ref[...] = a_ref[...] + b_ref[...]


def _add(a, b):
    return pl.pallas_call(
        _add_body,
        out_shape=jax.ShapeDtypeStruct((V, D), jnp.float32),
    )(a, b)


@jax.jit
def kernel(H, X_node):
    idx2d = jnp.pad(X_node.reshape(IDX_ROWS, BLK), ((0, 8), (0, 0)))
    idx3 = X_node.reshape(N // BN, 1, BN)
    zeros = jnp.zeros((V // NS, DC), jnp.float32)
    sc_part = _sc_aggr(H, idx2d, zeros)
    tc_part = _tc_matmul(H, idx3)
    return _add(tc_part, sc_part)
